# baseline (speedup 1.0000x reference)
import functools

import jax
import jax.numpy as jnp
import numpy as np
from jax import lax
from jax.experimental import pallas as pl
from jax.experimental.pallas import tpu as pltpu
from jax.experimental.pallas import tpu_sc as plsc

B, N, K = 8, 1024, 20
EMB = 512
ED = 4
BN_S = float(1.0 / np.sqrt(1.0 + 1e-5))
SLOPE = 0.2
BETA = 1.0
NEG = float("-inf")


def _knn_body(f_ref, idx_ref):
    f = f_ref[0]
    b = pl.program_id(0)
    sq = jnp.sum(f * f, axis=1, keepdims=True)
    inner = jax.lax.dot_general(f, f, (((1,), (1,)), ((), ())),
                                preferred_element_type=jnp.float32)
    sqr = jnp.sum(f * f, axis=1)[None, :]
    D = -((sq - 2.0 * inner) + sqr)
    iota_j = jax.lax.broadcasted_iota(jnp.int32, (N, N), 1)
    iota_t = jax.lax.broadcasted_iota(jnp.int32, (N, 32), 1)
    iota_r32 = jax.lax.broadcasted_iota(jnp.int32, (N, 32), 0)
    iota_r = jax.lax.broadcasted_iota(jnp.int32, (N, N), 0)

    def step(t, carry):
        D, idxs = carry
        m = jnp.max(D, axis=1, keepdims=True)
        cand = jnp.where(D == m, iota_j, N)
        am = jnp.min(cand, axis=1, keepdims=True)
        idxs = jnp.where(iota_t == t, am + b * N, idxs)
        D = jnp.where(iota_j == am, NEG, D)
        return D, idxs

    idxs0 = jnp.where(iota_t == 0, iota_r32 + b * N, 0)
    D = jnp.where(iota_j == iota_r, NEG, D)
    _, idxs = jax.lax.fori_loop(1, K, step, (D, idxs0))
    idx_ref[0] = idxs


def _knn_idx(f):
    Cin = f.shape[-1]
    nb = f.shape[0]
    return pl.pallas_call(
        _knn_body,
        grid=(nb,),
        in_specs=[pl.BlockSpec((1, N, Cin), lambda b: (b, 0, 0))],
        out_specs=pl.BlockSpec((1, N, 32), lambda b: (b, 0, 0)),
        out_shape=jax.ShapeDtypeStruct((nb, N, 32), jnp.int32),
    )(f)


def _sc_gather(idx_k_bn, table):
    BN, C = table.shape
    per_w = BN // 32
    mesh = plsc.VectorSubcoreMesh(core_axis_name="c", subcore_axis_name="s",
                                  num_cores=2, num_subcores=16)

    @functools.partial(
        pl.kernel,
        out_type=jax.ShapeDtypeStruct((K, BN, C), jnp.float32),
        mesh=mesh,
        compiler_params=pltpu.CompilerParams(use_tc_tiling_on_sc=False),
        scratch_types=[
            pltpu.VMEM((2 * K, 128), jnp.int32),
            pltpu.VMEM((128, C), jnp.float32),
            pltpu.VMEM((128, C), jnp.float32),
            pltpu.VMEM((128, C), jnp.float32),
            pltpu.VMEM((128, C), jnp.float32),
            pltpu.SemaphoreType.DMA,
            pltpu.SemaphoreType.DMA,
            pltpu.SemaphoreType.DMA,
            pltpu.SemaphoreType.DMA,
            pltpu.SemaphoreType.DMA,
            pltpu.SemaphoreType.DMA,
            pltpu.SemaphoreType.DMA,
            pltpu.SemaphoreType.DMA,
            pltpu.SemaphoreType.DMA,
        ],
    )
    def k(idx_hbm, tab_hbm, out_hbm, idx_v, rows_a, rows_b, rows_c, rows_d,
          isem, gsem_a, gsem_b, gsem_c, gsem_d, osem_a, osem_b, osem_c, osem_d):
        wid = lax.axis_index("s") * 2 + lax.axis_index("c")
        base = wid * per_w
        nsub = per_w // 128
        njob = K * nsub

        stage = []
        for t in range(K):
            for s2 in range(nsub):
                stage.append(pltpu.async_copy(
                    idx_hbm.at[t, pl.ds(base + s2 * 128, 128)],
                    idx_v.at[t * nsub + s2], isem))
        for d in stage:
            d.wait()

        NB = 4
        rows = (rows_a, rows_b, rows_c, rows_d)
        gsems = (gsem_a, gsem_b, gsem_c, gsem_d)
        osems = (osem_a, osem_b, osem_c, osem_d)
        copies = [None] * NB
        outs = [None] * NB
        for j in range(njob + 2):
            if j < njob:
                if j >= NB:
                    outs[j % NB].wait()
                copies[j % NB] = pltpu.async_copy(tab_hbm.at[idx_v.at[j]], rows[j % NB], gsems[j % NB])
            if 2 <= j < njob + 2:
                jj = j - 2
                copies[jj % NB].wait()
                t, s2 = divmod(jj, nsub)
                outs[jj % NB] = pltpu.async_copy(
                    rows[jj % NB], out_hbm.at[t, pl.ds(base + s2 * 128, 128)], osems[jj % NB])
        for r in range(NB):
            if outs[r] is not None:
                outs[r].wait()

    return k(idx_k_bn, table)


def _edge_from_gather_body(g_ref, f_ref, w2c_ref, sv_ref, out_ref, *, Cf):
    f = f_ref[0]
    Cout = out_ref.shape[-1]
    macc = jnp.full((N, Cout), NEG, dtype=jnp.float32)
    for t in range(K):
        g = g_ref[t, 0][:, :Cf]
        ef = jnp.concatenate([g - f, f], axis=1)
        y = jnp.dot(ef, w2c_ref[...], preferred_element_type=jnp.float32)
        y = y * sv_ref[...]
        y = jnp.where(y > 0, y, SLOPE * y)
        macc = jnp.maximum(macc, y)
    out_ref[0] = macc


def _edge_from_gather(G, f, w2c, sv):
    Cg = G.shape[-1]
    Cf = f.shape[-1]
    Cout = w2c.shape[-1]
    nb = f.shape[0]
    return pl.pallas_call(
        functools.partial(_edge_from_gather_body, Cf=Cf),
        grid=(nb,),
        in_specs=[
            pl.BlockSpec((K, 1, N, Cg), lambda b: (0, b, 0, 0)),
            pl.BlockSpec((1, N, Cf), lambda b: (b, 0, 0)),
            pl.BlockSpec((2 * Cf, Cout), lambda b: (0, 0)),
            pl.BlockSpec((1, Cout), lambda b: (0, 0)),
        ],
        out_specs=pl.BlockSpec((1, N, Cout), lambda b: (b, 0, 0)),
        out_shape=jax.ShapeDtypeStruct((nb, N, Cout), jnp.float32),
    )(G, f, w2c, sv)


def _edge_half(f, w2c, sv, table=None):
    nb = f.shape[0]
    idx = _knn_idx(f)
    idx_k = jnp.transpose(idx, (2, 0, 1)).reshape(32, nb * N)[:K]
    tab = f if table is None else table
    Ct = tab.shape[-1]
    G = _sc_gather(idx_k, tab.reshape(nb * N, Ct)).reshape(K, nb, N, Ct)
    return _edge_from_gather(G, f, w2c, sv)


def _edge_layer(f, w2c, sv, table=None):
    h = f.shape[0] // 2
    ta = None if table is None else table[:h]
    tb = None if table is None else table[h:]
    xa = _edge_half(f[:h], w2c, sv, table=ta)
    xb = _edge_half(f[h:], w2c, sv, table=tb)
    return jnp.concatenate([xa, xb], axis=0)


def _conv5_pool_body(x1_ref, x2_ref, x3_ref, x4_ref, w_ref, sv_ref, out_ref):
    hcat = jnp.concatenate([x1_ref[0], x2_ref[0], x3_ref[0], x4_ref[0]], axis=1)
    hh = jnp.dot(hcat, w_ref[...], preferred_element_type=jnp.float32) * sv_ref[...]
    y = jnp.where(hh > 0, hh, SLOPE * hh)
    out_ref[0, 0] = jnp.max(y, axis=0)


def _conv5_pool(x1, x2, x3, x4, w5t, sv5):
    return pl.pallas_call(
        _conv5_pool_body,
        grid=(B,),
        in_specs=[
            pl.BlockSpec((1, N, 64), lambda b: (b, 0, 0)),
            pl.BlockSpec((1, N, 64), lambda b: (b, 0, 0)),
            pl.BlockSpec((1, N, 128), lambda b: (b, 0, 0)),
            pl.BlockSpec((1, N, 256), lambda b: (b, 0, 0)),
            pl.BlockSpec((512, EMB), lambda b: (0, 0)),
            pl.BlockSpec((1, EMB), lambda b: (0, 0)),
        ],
        out_specs=pl.BlockSpec((1, 1, EMB), lambda b: (b, 0, 0)),
        out_shape=jax.ShapeDtypeStruct((B, 1, EMB), jnp.float32),
    )(x1, x2, x3, x4, w5t, sv5)


def _vq_body(hf_ref, wq_ref, bq_ref, cb_ref, cbt_ref, wpq_ref, bpq_ref,
             q_ref, loss_ref):
    R = B * EMB
    zf = hf_ref[...] * wq_ref[...] + bq_ref[...]
    cbt = cbt_ref[...]
    rs = jnp.sum(zf * zf, axis=1, keepdims=True)
    cs = jnp.sum(cbt * cbt, axis=0, keepdims=True)
    cross = jnp.dot(2.0 * zf, cbt, preferred_element_type=jnp.float32)
    dist = (rs + cs) - cross
    iota_j = jax.lax.broadcasted_iota(jnp.int32, (R, EMB), 1)
    m = jnp.min(dist, axis=1, keepdims=True)
    cand = jnp.where(dist == m, iota_j, EMB)
    am = jnp.min(cand, axis=1, keepdims=True)
    oh = (iota_j == am).astype(jnp.float32)
    zq = jnp.dot(oh, cb_ref[...], precision=jax.lax.Precision.HIGHEST,
                 preferred_element_type=jnp.float32)
    diff = zq - zf
    msq = jnp.sum(diff * diff) / (R * ED)
    loss_ref[...] = jnp.reshape(BETA * msq + msq, (1, 1))
    zqf = zf + (zq - zf)
    q_ref[...] = jnp.dot(zqf, wpq_ref[...], preferred_element_type=jnp.float32) + bpq_ref[...]


def _vq(hf, wq, bq2, cb, cbt, wpq, bpq2):
    R = B * EMB
    return pl.pallas_call(
        _vq_body,
        out_shape=(jax.ShapeDtypeStruct((R, 1), jnp.float32),
                   jax.ShapeDtypeStruct((1, 1), jnp.float32)),
    )(hf, wq, bq2, cb, cbt, wpq, bpq2)


def _dec_body(q_ref, w1_ref, b1_ref, w2_ref, b2_ref, w3_ref, b3_ref, out_ref):
    h1 = jnp.dot(q_ref[...], w1_ref[...], preferred_element_type=jnp.float32) + b1_ref[...]
    h1 = jnp.maximum(h1, 0.0)
    h2 = jnp.dot(h1, w2_ref[...], preferred_element_type=jnp.float32) + b2_ref[...]
    h2 = jnp.maximum(h2, 0.0)
    out_ref[...] = jnp.dot(h2, w3_ref[...], preferred_element_type=jnp.float32) + b3_ref[...]


def _decode(q8, Wd1, bd1, Wd2, bd2, Wd3, bd3):
    return pl.pallas_call(
        _dec_body,
        out_shape=jax.ShapeDtypeStruct((B, 3 * N), jnp.float32),
    )(q8, Wd1, bd1.reshape(1, -1), Wd2, bd2.reshape(1, -1), Wd3, bd3.reshape(1, -1))


@jax.jit
def kernel(x, W1, g1, W2, g2, W3, g3, W4, g4, W5, g5, Wq, bq, codebook,
           Wpq, bpq, Wd1, bd1, Wd2, bd2, Wd3, bd3):
    feat = jnp.transpose(x, (0, 2, 1))

    feat16 = jnp.pad(feat, ((0, 0), (0, 0), (0, 13)))
    x1 = _edge_layer(feat, jnp.transpose(W1), (g1 * BN_S).reshape(1, -1),
                     table=feat16)
    x2 = _edge_layer(x1, jnp.transpose(W2), (g2 * BN_S).reshape(1, -1))
    x3 = _edge_layer(x2, jnp.transpose(W3), (g3 * BN_S).reshape(1, -1))
    x4 = _edge_layer(x3, jnp.transpose(W4), (g4 * BN_S).reshape(1, -1))

    hmax = _conv5_pool(x1, x2, x3, x4, jnp.transpose(W5),
                       (g5 * BN_S).reshape(1, -1))

    hf = hmax.reshape(B * EMB, 1)
    q, loss = _vq(hf, Wq, bq.reshape(1, ED), codebook, codebook.T,
                  Wpq, bpq.reshape(1, 1))
    q8 = q.reshape(B, EMB)
    dec = _decode(q8, Wd1, bd1, Wd2, bd2, Wd3, bd3).reshape(B, 3, N)
    return dec, loss[0, 0]


# speedup vs baseline: 1.0047x; 1.0047x over previous
import functools

import jax
import jax.numpy as jnp
import numpy as np
from jax import lax
from jax.experimental import pallas as pl
from jax.experimental.pallas import tpu as pltpu
from jax.experimental.pallas import tpu_sc as plsc

B, N, K = 8, 1024, 20
EMB = 512
ED = 4
BN_S = float(1.0 / np.sqrt(1.0 + 1e-5))
SLOPE = 0.2
BETA = 1.0
NEG = float("-inf")


def _knn_body(f_ref, idx_ref):
    f = f_ref[0]
    b = pl.program_id(0)
    sq = jnp.sum(f * f, axis=1, keepdims=True)
    inner = jax.lax.dot_general(f, f, (((1,), (1,)), ((), ())),
                                preferred_element_type=jnp.float32)
    sqr = jnp.sum(f * f, axis=1)[None, :]
    D = -((sq - 2.0 * inner) + sqr)
    iota_j = jax.lax.broadcasted_iota(jnp.int32, (N, N), 1)
    iota_t = jax.lax.broadcasted_iota(jnp.int32, (N, 32), 1)
    iota_r32 = jax.lax.broadcasted_iota(jnp.int32, (N, 32), 0)
    iota_r = jax.lax.broadcasted_iota(jnp.int32, (N, N), 0)

    def step(t, carry):
        D, idxs = carry
        m = jnp.max(D, axis=1, keepdims=True)
        cand = jnp.where(D == m, iota_j, N)
        am = jnp.min(cand, axis=1, keepdims=True)
        idxs = jnp.where(iota_t == t, am + b * N, idxs)
        D = jnp.where(iota_j == am, NEG, D)
        return D, idxs

    idxs0 = jnp.where(iota_t == 0, iota_r32 + b * N, 0)
    D = jnp.where(iota_j == iota_r, NEG, D)
    _, idxs = jax.lax.fori_loop(1, K, step, (D, idxs0))
    idx_ref[0] = idxs


def _knn_idx(f):
    Cin = f.shape[-1]
    return pl.pallas_call(
        _knn_body,
        grid=(B,),
        in_specs=[pl.BlockSpec((1, N, Cin), lambda b: (b, 0, 0))],
        out_specs=pl.BlockSpec((1, N, 32), lambda b: (b, 0, 0)),
        out_shape=jax.ShapeDtypeStruct((B, N, 32), jnp.int32),
    )(f)


def _sc_gather(idx_k_bn, table):
    BN, C = table.shape
    per_w = BN // 32
    mesh = plsc.VectorSubcoreMesh(core_axis_name="c", subcore_axis_name="s",
                                  num_cores=2, num_subcores=16)

    @functools.partial(
        pl.kernel,
        out_type=jax.ShapeDtypeStruct((K, BN, C), jnp.float32),
        mesh=mesh,
        compiler_params=pltpu.CompilerParams(use_tc_tiling_on_sc=False),
        scratch_types=[
            pltpu.VMEM((2 * K, 128), jnp.int32),
            pltpu.VMEM((128, C), jnp.float32),
            pltpu.VMEM((128, C), jnp.float32),
            pltpu.VMEM((128, C), jnp.float32),
            pltpu.VMEM((128, C), jnp.float32),
            pltpu.SemaphoreType.DMA,
            pltpu.SemaphoreType.DMA,
            pltpu.SemaphoreType.DMA,
            pltpu.SemaphoreType.DMA,
            pltpu.SemaphoreType.DMA,
            pltpu.SemaphoreType.DMA,
            pltpu.SemaphoreType.DMA,
            pltpu.SemaphoreType.DMA,
            pltpu.SemaphoreType.DMA,
        ],
    )
    def k(idx_hbm, tab_hbm, out_hbm, idx_v, rows_a, rows_b, rows_c, rows_d,
          isem, gsem_a, gsem_b, gsem_c, gsem_d, osem_a, osem_b, osem_c, osem_d):
        wid = lax.axis_index("s") * 2 + lax.axis_index("c")
        base = wid * per_w
        nsub = per_w // 128
        njob = K * nsub

        stage = []
        for t in range(K):
            for s2 in range(nsub):
                stage.append(pltpu.async_copy(
                    idx_hbm.at[t, pl.ds(base + s2 * 128, 128)],
                    idx_v.at[t * nsub + s2], isem))
        for d in stage:
            d.wait()

        NB = 4
        rows = (rows_a, rows_b, rows_c, rows_d)
        gsems = (gsem_a, gsem_b, gsem_c, gsem_d)
        osems = (osem_a, osem_b, osem_c, osem_d)
        copies = [None] * NB
        outs = [None] * NB
        for j in range(njob + 2):
            if j < njob:
                if j >= NB:
                    outs[j % NB].wait()
                copies[j % NB] = pltpu.async_copy(tab_hbm.at[idx_v.at[j]], rows[j % NB], gsems[j % NB])
            if 2 <= j < njob + 2:
                jj = j - 2
                copies[jj % NB].wait()
                t, s2 = divmod(jj, nsub)
                outs[jj % NB] = pltpu.async_copy(
                    rows[jj % NB], out_hbm.at[t, pl.ds(base + s2 * 128, 128)], osems[jj % NB])
        for r in range(NB):
            if outs[r] is not None:
                outs[r].wait()

    return k(idx_k_bn, table)


def _edge_from_gather_body(g_ref, f_ref, w2c_ref, sv_ref, out_ref, *, Cf):
    f = f_ref[0]
    Cout = out_ref.shape[-1]
    macc = jnp.full((N, Cout), NEG, dtype=jnp.float32)
    for t in range(K):
        g = g_ref[t, 0][:, :Cf]
        ef = jnp.concatenate([g - f, f], axis=1)
        y = jnp.dot(ef, w2c_ref[...], preferred_element_type=jnp.float32)
        y = y * sv_ref[...]
        y = jnp.where(y > 0, y, SLOPE * y)
        macc = jnp.maximum(macc, y)
    out_ref[0] = macc


def _edge_from_gather(G, f, w2c, sv):
    Cg = G.shape[-1]
    Cf = f.shape[-1]
    Cout = w2c.shape[-1]
    return pl.pallas_call(
        functools.partial(_edge_from_gather_body, Cf=Cf),
        grid=(B,),
        in_specs=[
            pl.BlockSpec((K, 1, N, Cg), lambda b: (0, b, 0, 0)),
            pl.BlockSpec((1, N, Cf), lambda b: (b, 0, 0)),
            pl.BlockSpec((2 * Cf, Cout), lambda b: (0, 0)),
            pl.BlockSpec((1, Cout), lambda b: (0, 0)),
        ],
        out_specs=pl.BlockSpec((1, N, Cout), lambda b: (b, 0, 0)),
        out_shape=jax.ShapeDtypeStruct((B, N, Cout), jnp.float32),
    )(G, f, w2c, sv)


def _edge_layer(f, w2c, sv, table=None):
    Cf = f.shape[-1]
    idx = _knn_idx(f)
    idx_k = jnp.transpose(idx, (2, 0, 1)).reshape(32, B * N)[:K]
    tab = f if table is None else table
    Ct = tab.shape[-1]
    G = _sc_gather(idx_k, tab.reshape(B * N, Ct)).reshape(K, B, N, Ct)
    return _edge_from_gather(G, f, w2c, sv)


def _conv5_pool_body(x1_ref, x2_ref, x3_ref, x4_ref, w_ref, sv_ref, out_ref):
    hcat = jnp.concatenate([x1_ref[0], x2_ref[0], x3_ref[0], x4_ref[0]], axis=1)
    hh = jnp.dot(hcat, w_ref[...], preferred_element_type=jnp.float32) * sv_ref[...]
    y = jnp.where(hh > 0, hh, SLOPE * hh)
    out_ref[0, 0] = jnp.max(y, axis=0)


def _conv5_pool(x1, x2, x3, x4, w5t, sv5):
    return pl.pallas_call(
        _conv5_pool_body,
        grid=(B,),
        in_specs=[
            pl.BlockSpec((1, N, 64), lambda b: (b, 0, 0)),
            pl.BlockSpec((1, N, 64), lambda b: (b, 0, 0)),
            pl.BlockSpec((1, N, 128), lambda b: (b, 0, 0)),
            pl.BlockSpec((1, N, 256), lambda b: (b, 0, 0)),
            pl.BlockSpec((512, EMB), lambda b: (0, 0)),
            pl.BlockSpec((1, EMB), lambda b: (0, 0)),
        ],
        out_specs=pl.BlockSpec((1, 1, EMB), lambda b: (b, 0, 0)),
        out_shape=jax.ShapeDtypeStruct((B, 1, EMB), jnp.float32),
    )(x1, x2, x3, x4, w5t, sv5)


def _vq_body(hf_ref, wq_ref, bq_ref, cb_ref, cbt_ref, wpq_ref, bpq_ref,
             q_ref, loss_ref):
    R = B * EMB
    zf = hf_ref[...] * wq_ref[...] + bq_ref[...]
    cbt = cbt_ref[...]
    rs = jnp.sum(zf * zf, axis=1, keepdims=True)
    cs = jnp.sum(cbt * cbt, axis=0, keepdims=True)
    cross = jnp.dot(2.0 * zf, cbt, preferred_element_type=jnp.float32)
    dist = (rs + cs) - cross
    iota_j = jax.lax.broadcasted_iota(jnp.int32, (R, EMB), 1)
    m = jnp.min(dist, axis=1, keepdims=True)
    cand = jnp.where(dist == m, iota_j, EMB)
    am = jnp.min(cand, axis=1, keepdims=True)
    oh = (iota_j == am).astype(jnp.float32)
    zq = jnp.dot(oh, cb_ref[...], precision=jax.lax.Precision.HIGHEST,
                 preferred_element_type=jnp.float32)
    diff = zq - zf
    msq = jnp.sum(diff * diff) / (R * ED)
    loss_ref[...] = jnp.reshape(BETA * msq + msq, (1, 1))
    zqf = zf + (zq - zf)
    q_ref[...] = jnp.dot(zqf, wpq_ref[...], preferred_element_type=jnp.float32) + bpq_ref[...]


def _vq(hf, wq, bq2, cb, cbt, wpq, bpq2):
    R = B * EMB
    return pl.pallas_call(
        _vq_body,
        out_shape=(jax.ShapeDtypeStruct((R, 1), jnp.float32),
                   jax.ShapeDtypeStruct((1, 1), jnp.float32)),
    )(hf, wq, bq2, cb, cbt, wpq, bpq2)


def _dec_body(q_ref, w1_ref, b1_ref, w2_ref, b2_ref, w3_ref, b3_ref, out_ref):
    h1 = jnp.dot(q_ref[...], w1_ref[...], preferred_element_type=jnp.float32) + b1_ref[...]
    h1 = jnp.maximum(h1, 0.0)
    h2 = jnp.dot(h1, w2_ref[...], preferred_element_type=jnp.float32) + b2_ref[...]
    h2 = jnp.maximum(h2, 0.0)
    out_ref[...] = jnp.dot(h2, w3_ref[...], preferred_element_type=jnp.float32) + b3_ref[...]


def _decode(q8, Wd1, bd1, Wd2, bd2, Wd3, bd3):
    return pl.pallas_call(
        _dec_body,
        out_shape=jax.ShapeDtypeStruct((B, 3 * N), jnp.float32),
    )(q8, Wd1, bd1.reshape(1, -1), Wd2, bd2.reshape(1, -1), Wd3, bd3.reshape(1, -1))


@jax.jit
def kernel(x, W1, g1, W2, g2, W3, g3, W4, g4, W5, g5, Wq, bq, codebook,
           Wpq, bpq, Wd1, bd1, Wd2, bd2, Wd3, bd3):
    feat = jnp.transpose(x, (0, 2, 1))

    feat16 = jnp.pad(feat, ((0, 0), (0, 0), (0, 13)))
    x1 = _edge_layer(feat, jnp.transpose(W1), (g1 * BN_S).reshape(1, -1),
                     table=feat16)
    x2 = _edge_layer(x1, jnp.transpose(W2), (g2 * BN_S).reshape(1, -1))
    x3 = _edge_layer(x2, jnp.transpose(W3), (g3 * BN_S).reshape(1, -1))
    x4 = _edge_layer(x3, jnp.transpose(W4), (g4 * BN_S).reshape(1, -1))

    hmax = _conv5_pool(x1, x2, x3, x4, jnp.transpose(W5),
                       (g5 * BN_S).reshape(1, -1))

    hf = hmax.reshape(B * EMB, 1)
    q, loss = _vq(hf, Wq, bq.reshape(1, ED), codebook, codebook.T,
                  Wpq, bpq.reshape(1, 1))
    q8 = q.reshape(B, EMB)
    dec = _decode(q8, Wd1, bd1, Wd2, bd2, Wd3, bd3).reshape(B, 3, N)
    return dec, loss[0, 0]
